# baseline (speedup 1.0000x reference)
import jax
import jax.numpy as jnp
from jax import lax
from jax.experimental import pallas as pl
from jax.experimental.pallas import tpu as pltpu
from jax.experimental.pallas import tpu_sc as plsc

N_NODES = 16384
DIM = 32
HALF = DIM // 2
NUM_CORES = 2
NUM_SUBCORES = 16
NUM_WORKERS = NUM_CORES * NUM_SUBCORES
B_PER_W = N_NODES // NUM_WORKERS
PACK = 128 // HALF
ROWS_PACKED = B_PER_W // PACK


def _half_body(ids_hbm, cid_hbm, table_hbm, nodes_out, char_out,
               idx_v, cidx_v, rows_v, crow_v, sem, csem, osem):
    wid = lax.axis_index("s") * NUM_CORES + lax.axis_index("c")
    base = wid * B_PER_W

    pltpu.sync_copy(ids_hbm.at[pl.ds(base, B_PER_W)], idx_v)

    @pl.when(wid == 0)
    def _():
        pltpu.sync_copy(cid_hbm, cidx_v.at[pl.ds(0, 1)])
        cv = cidx_v[...]
        off = pl.multiple_of(cv[0] * HALF, 8)
        pltpu.async_copy(table_hbm.at[pl.ds(off, HALF)], crow_v, csem)

    def issue(g, _):
        v = idx_v[pl.ds(g * 16, 16)]
        for l in range(16):
            j = g * 16 + l
            off = pl.multiple_of(v[l] * HALF, 8)
            pltpu.async_copy(
                table_hbm.at[pl.ds(off, HALF)],
                rows_v.at[pl.ds(j * HALF, HALF)],
                sem,
            )
        return ()

    lax.fori_loop(0, B_PER_W // 16, issue, ())

    def drain(j, _):
        pltpu.make_async_copy(
            table_hbm.at[pl.ds(0, HALF)],
            rows_v.at[pl.ds(0, HALF)],
            sem,
        ).wait()
        return ()

    lax.fori_loop(0, B_PER_W, drain, (), unroll=8)

    pltpu.async_copy(
        rows_v, nodes_out.at[pl.ds(base * HALF, B_PER_W * HALF)], osem
    ).wait()

    @pl.when(wid == 0)
    def _():
        pltpu.make_async_copy(table_hbm.at[pl.ds(0, HALF)], crow_v, csem).wait()
        pltpu.sync_copy(crow_v, char_out)


def _make_half():
    mesh = plsc.VectorSubcoreMesh(core_axis_name="c", subcore_axis_name="s")
    return pl.kernel(
        _half_body,
        mesh=mesh,
        out_type=(
            jax.ShapeDtypeStruct((N_NODES * HALF,), jnp.float32),
            jax.ShapeDtypeStruct((HALF,), jnp.float32),
        ),
        scratch_types=[
            pltpu.VMEM((B_PER_W,), jnp.int32),
            pltpu.VMEM((16,), jnp.int32),
            pltpu.VMEM((B_PER_W * HALF,), jnp.float32),
            pltpu.VMEM((HALF,), jnp.float32),
            pltpu.SemaphoreType.DMA,
            pltpu.SemaphoreType.DMA,
            pltpu.SemaphoreType.DMA,
        ],
        compiler_params=pltpu.CompilerParams(use_tc_tiling_on_sc=True),
    )


def kernel(node_name_ids, char_id, object_embedding):
    ids = node_name_ids.astype(jnp.int32)
    cid = char_id.astype(jnp.int32)
    t_left = object_embedding[:, :HALF].reshape(-1)
    t_right = object_embedding[:, HALF:].reshape(-1)
    nl, cl = _make_half()(ids, cid, t_left)
    nr, cr = _make_half()(ids, cid, t_right)
    nodes = jnp.concatenate(
        [nl.reshape(N_NODES, HALF), nr.reshape(N_NODES, HALF)], axis=1
    )
    char = jnp.concatenate([cl, cr]).reshape(1, DIM)
    return (nodes, char)


# speedup vs baseline: 2.8028x; 2.8028x over previous
import jax
import jax.numpy as jnp
from jax import lax
from jax.experimental import pallas as pl
from jax.experimental.pallas import tpu as pltpu
from jax.experimental.pallas import tpu_sc as plsc

N_NODES = 16384
DIM = 32
NUM_CORES = 2
NUM_SUBCORES = 16
NUM_WORKERS = NUM_CORES * NUM_SUBCORES
B_PER_W = N_NODES // NUM_WORKERS


def _gather_body(ids_hbm, cid_hbm, table_hbm, nodes_out, char_out,
                 idx_v, cidx_v, rows_v, crow_v, sem, csem, osem):
    wid = lax.axis_index("s") * NUM_CORES + lax.axis_index("c")
    base = wid * B_PER_W

    pltpu.sync_copy(ids_hbm.at[pl.ds(base, B_PER_W)], idx_v)

    @pl.when(wid == 0)
    def _():
        pltpu.sync_copy(cid_hbm, cidx_v.at[pl.ds(0, 1)])
        cv = cidx_v[...]
        pltpu.async_copy(table_hbm.at[pl.ds(cv[0], 1), :], crow_v, csem)

    def issue(g, _):
        v = idx_v[pl.ds(g * 16, 16)]
        for l in range(16):
            pltpu.async_copy(
                table_hbm.at[pl.ds(v[l], 1), :],
                rows_v.at[pl.ds(g * 16 + l, 1), :],
                sem,
            )
        return ()

    lax.fori_loop(0, B_PER_W // 16, issue, ())

    def drain(j, _):
        pltpu.make_async_copy(
            table_hbm.at[pl.ds(0, 1), :],
            rows_v.at[pl.ds(j, 1), :],
            sem,
        ).wait()
        return ()

    lax.fori_loop(0, B_PER_W, drain, (), unroll=8)

    pltpu.async_copy(rows_v, nodes_out.at[pl.ds(base, B_PER_W)], osem).wait()

    @pl.when(wid == 0)
    def _():
        pltpu.make_async_copy(table_hbm.at[pl.ds(0, 1), :], crow_v, csem).wait()
        pltpu.sync_copy(crow_v, char_out)


def kernel(node_name_ids, char_id, object_embedding):
    mesh = plsc.VectorSubcoreMesh(core_axis_name="c", subcore_axis_name="s")
    f = pl.kernel(
        _gather_body,
        mesh=mesh,
        out_type=(
            jax.ShapeDtypeStruct((N_NODES, DIM), jnp.float32),
            jax.ShapeDtypeStruct((1, DIM), jnp.float32),
        ),
        scratch_types=[
            pltpu.VMEM((B_PER_W,), jnp.int32),
            pltpu.VMEM((16,), jnp.int32),
            pltpu.VMEM((B_PER_W, DIM), jnp.float32),
            pltpu.VMEM((1, DIM), jnp.float32),
            pltpu.SemaphoreType.DMA,
            pltpu.SemaphoreType.DMA,
            pltpu.SemaphoreType.DMA,
        ],
        compiler_params=pltpu.CompilerParams(use_tc_tiling_on_sc=True),
    )
    node_embeddings, char_embedding = f(
        node_name_ids.astype(jnp.int32),
        char_id.astype(jnp.int32),
        object_embedding,
    )
    return (node_embeddings, char_embedding)
